# Initial kernel scaffold; baseline (speedup 1.0000x reference)
import jax
import jax.numpy as jnp
from jax.experimental import pallas as pl


def kernel(x, tables, W, b):
    raise NotImplementedError("write your pallas kernel here")


# speedup vs baseline: 7.4284x; 7.4284x over previous
import functools

import jax
import jax.numpy as jnp
from jax import lax
from jax.experimental import pallas as pl
from jax.experimental.pallas import tpu as pltpu
from jax.experimental.pallas import tpu_sc as plsc

B = 16384
NUM_FIELDS = 26
VOCAB = 100000
PER_FIELD_DIM = 32
HIDDEN = NUM_FIELDS * PER_FIELD_DIM
OUT_DIM = 128

NC = 2
NS = 16
NW = NC * NS
IDX_PER_TILE = B * NUM_FIELDS // NW
K = 128
STEPS = IDX_PER_TILE // K


def _sc_gather(xg_flat, tables_flat):
    mesh = plsc.VectorSubcoreMesh(core_axis_name="c", subcore_axis_name="s")

    @functools.partial(
        pl.kernel,
        mesh=mesh,
        out_type=jax.ShapeDtypeStruct((B * NUM_FIELDS, PER_FIELD_DIM),
                                      jnp.float32),
        scratch_types=[
            pltpu.VMEM((K,), jnp.int32),
            pltpu.VMEM((K, PER_FIELD_DIM), jnp.float32),
            pltpu.SemaphoreType.DMA,
        ],
        compiler_params=pltpu.CompilerParams(use_tc_tiling_on_sc=False),
    )
    def k(xg_hbm, tab_hbm, out_hbm, idx_v, rows_v, sem):
        wid = lax.axis_index("s") * NC + lax.axis_index("c")
        base = wid * IDX_PER_TILE

        def body(j, carry):
            off = pl.multiple_of(base + j * K, K)
            pltpu.sync_copy(xg_hbm.at[pl.ds(off, K)], idx_v)
            pltpu.async_copy(tab_hbm.at[idx_v], rows_v, sem).wait()
            pltpu.sync_copy(rows_v, out_hbm.at[pl.ds(off, K)])
            return carry

        lax.fori_loop(0, STEPS, body, 0)

    return k(xg_flat, tables_flat)


def _tc_head(emb, wt, b2):
    bm = 2048

    def body(e_ref, w_ref, b_ref, o_ref):
        h = jnp.maximum(e_ref[...], 0.0)
        o_ref[...] = (
            jnp.dot(h, w_ref[...], preferred_element_type=jnp.float32)
            + b_ref[...]
        )

    return pl.pallas_call(
        body,
        grid=(B // bm,),
        in_specs=[
            pl.BlockSpec((bm, HIDDEN), lambda i: (i, 0)),
            pl.BlockSpec((HIDDEN, OUT_DIM), lambda i: (0, 0)),
            pl.BlockSpec((1, OUT_DIM), lambda i: (0, 0)),
        ],
        out_specs=pl.BlockSpec((bm, OUT_DIM), lambda i: (i, 0)),
        out_shape=jax.ShapeDtypeStruct((B, OUT_DIM), jnp.float32),
    )(emb, wt, b2)


def kernel(x, tables, W, b):
    offs = (jnp.arange(NUM_FIELDS, dtype=jnp.int32) * VOCAB)[None, :]
    xg_flat = (x + offs).reshape(B * NUM_FIELDS)
    tables_flat = tables.reshape(NUM_FIELDS * VOCAB, PER_FIELD_DIM)
    emb = _sc_gather(xg_flat, tables_flat).reshape(B, HIDDEN)
    return _tc_head(emb, W.T, b.reshape(1, OUT_DIM))


# speedup vs baseline: 8.1140x; 1.0923x over previous
import functools

import jax
import jax.numpy as jnp
from jax import lax
from jax.experimental import pallas as pl
from jax.experimental.pallas import tpu as pltpu
from jax.experimental.pallas import tpu_sc as plsc

B = 16384
NUM_FIELDS = 26
VOCAB = 100000
PER_FIELD_DIM = 32
HIDDEN = NUM_FIELDS * PER_FIELD_DIM
OUT_DIM = 128

NC = 2
NS = 16
NW = NC * NS
IDX_PER_TILE = B * NUM_FIELDS // NW
CHUNK = 1664
NCHUNK = IDX_PER_TILE // CHUNK


def _sc_gather(xg_flat, tables_flat):
    mesh = plsc.VectorSubcoreMesh(core_axis_name="c", subcore_axis_name="s")

    @functools.partial(
        pl.kernel,
        mesh=mesh,
        out_type=jax.ShapeDtypeStruct((B * NUM_FIELDS, PER_FIELD_DIM),
                                      jnp.float32),
        scratch_types=[
            pltpu.VMEM((IDX_PER_TILE,), jnp.int32),
            pltpu.VMEM((CHUNK, PER_FIELD_DIM), jnp.float32),
            pltpu.VMEM((CHUNK, PER_FIELD_DIM), jnp.float32),
            pltpu.SemaphoreType.DMA,
            pltpu.SemaphoreType.DMA,
            pltpu.SemaphoreType.DMA,
            pltpu.SemaphoreType.DMA,
        ],
        compiler_params=pltpu.CompilerParams(use_tc_tiling_on_sc=False),
    )
    def k(xg_hbm, tab_hbm, out_hbm, idx_v, rows0, rows1, g0, g1, w0, w1):
        wid = lax.axis_index("s") * NC + lax.axis_index("c")
        base = wid * IDX_PER_TILE
        pltpu.sync_copy(xg_hbm.at[pl.ds(base, IDX_PER_TILE)], idx_v)

        rows = (rows0, rows1)
        gsem = (g0, g1)
        wsem = (w0, w1)
        gathers = [None, None]
        writes = [None, None]

        def fire_gather(j, b):
            return pltpu.async_copy(
                tab_hbm.at[idx_v.at[pl.ds(j * CHUNK, CHUNK)]],
                rows[b], gsem[b])

        def fire_write(j, b):
            return pltpu.async_copy(
                rows[b], out_hbm.at[pl.ds(base + j * CHUNK, CHUNK)],
                wsem[b])

        for j in range(NCHUNK):
            b = j & 1
            if writes[b] is not None:
                writes[b].wait()
            gathers[b] = fire_gather(j, b)
            if j >= 1:
                gathers[1 - b].wait()
                writes[1 - b] = fire_write(j - 1, 1 - b)
        last = (NCHUNK - 1) & 1
        gathers[last].wait()
        writes[last] = fire_write(NCHUNK - 1, last)
        writes[1 - last].wait()
        writes[last].wait()

    return k(xg_flat, tables_flat)


def _tc_head(emb, wt, b2):
    bm = 2048

    def body(e_ref, w_ref, b_ref, o_ref):
        h = jnp.maximum(e_ref[...], 0.0)
        o_ref[...] = (
            jnp.dot(h, w_ref[...], preferred_element_type=jnp.float32)
            + b_ref[...]
        )

    return pl.pallas_call(
        body,
        grid=(B // bm,),
        in_specs=[
            pl.BlockSpec((bm, HIDDEN), lambda i: (i, 0)),
            pl.BlockSpec((HIDDEN, OUT_DIM), lambda i: (0, 0)),
            pl.BlockSpec((1, OUT_DIM), lambda i: (0, 0)),
        ],
        out_specs=pl.BlockSpec((bm, OUT_DIM), lambda i: (i, 0)),
        out_shape=jax.ShapeDtypeStruct((B, OUT_DIM), jnp.float32),
    )(emb, wt, b2)


def kernel(x, tables, W, b):
    offs = (jnp.arange(NUM_FIELDS, dtype=jnp.int32) * VOCAB)[None, :]
    xg_flat = (x + offs).reshape(B * NUM_FIELDS)
    tables_flat = tables.reshape(NUM_FIELDS * VOCAB, PER_FIELD_DIM)
    emb = _sc_gather(xg_flat, tables_flat).reshape(B, HIDDEN)
    return _tc_head(emb, W.T, b.reshape(1, OUT_DIM))


# speedup vs baseline: 27.6794x; 3.4113x over previous
import functools

import jax
import jax.numpy as jnp
from jax import lax
from jax.experimental import pallas as pl
from jax.experimental.pallas import tpu as pltpu
from jax.experimental.pallas import tpu_sc as plsc

B = 16384
NUM_FIELDS = 26
VOCAB = 100000
PER_FIELD_DIM = 32
HIDDEN = NUM_FIELDS * PER_FIELD_DIM
OUT_DIM = 128

NC = 2
NS = 16
NW = NC * NS
UNITS_PER_TILE = HIDDEN // NW
HALF_B = B // 2


def _sc_gather(xt_flat, tables_t):
    mesh = plsc.VectorSubcoreMesh(core_axis_name="c", subcore_axis_name="s")

    @functools.partial(
        pl.kernel,
        mesh=mesh,
        out_type=jax.ShapeDtypeStruct((HIDDEN, B), jnp.float32),
        scratch_types=[
            pltpu.VMEM((VOCAB,), jnp.float32),
            pltpu.VMEM((HALF_B,), jnp.int32),
            pltpu.VMEM((HALF_B,), jnp.float32),
            pltpu.VMEM((HALF_B,), jnp.float32),
            pltpu.SemaphoreType.DMA,
            pltpu.SemaphoreType.DMA,
            pltpu.SemaphoreType.DMA,
        ],
        compiler_params=pltpu.CompilerParams(needs_layout_passes=False),
    )
    def k(xt_hbm, tab_hbm, out_hbm, slice_v, idx_v, ob0, ob1, ssem, os0, os1):
        wid = lax.axis_index("s") * NC + lax.axis_index("c")
        obufs = (ob0, ob1)
        osems = (os0, os1)
        writes = [None, None]

        def gather_half(ob):
            def gbody(j, carry):
                ids = idx_v[pl.ds(j * 16, 16)]
                ob[pl.ds(j * 16, 16)] = plsc.load_gather(slice_v, [ids])
                return carry

            lax.fori_loop(0, HALF_B // 16, gbody, 0)

        for u in range(UNITS_PER_TILE):
            g = wid * UNITS_PER_TILE + u
            f = g // PER_FIELD_DIM
            d = g % PER_FIELD_DIM
            slice_cp = pltpu.async_copy(tab_hbm.at[f, d, :], slice_v, ssem)
            for h in range(2):
                pltpu.sync_copy(
                    xt_hbm.at[pl.ds(f * B + h * HALF_B, HALF_B)], idx_v)
                if h == 0:
                    slice_cp.wait()
                if writes[h] is not None:
                    writes[h].wait()
                gather_half(obufs[h])
                writes[h] = pltpu.async_copy(
                    obufs[h],
                    out_hbm.at[g, pl.ds(h * HALF_B, HALF_B)],
                    osems[h])
        writes[0].wait()
        writes[1].wait()

    return k(xt_flat, tables_t)


def _tc_head(h1t, wt, b2):
    bm = 1024

    def body(e_ref, w_ref, b_ref, o_ref):
        h = jnp.maximum(e_ref[...], 0.0)
        o_ref[...] = (
            jax.lax.dot_general(
                h, w_ref[...],
                dimension_numbers=(((0,), (0,)), ((), ())),
                preferred_element_type=jnp.float32)
            + b_ref[...]
        )

    return pl.pallas_call(
        body,
        grid=(B // bm,),
        in_specs=[
            pl.BlockSpec((HIDDEN, bm), lambda i: (0, i)),
            pl.BlockSpec((HIDDEN, OUT_DIM), lambda i: (0, 0)),
            pl.BlockSpec((1, OUT_DIM), lambda i: (0, 0)),
        ],
        out_specs=pl.BlockSpec((bm, OUT_DIM), lambda i: (i, 0)),
        out_shape=jax.ShapeDtypeStruct((B, OUT_DIM), jnp.float32),
    )(h1t, wt, b2)


def kernel(x, tables, W, b):
    xt_flat = jnp.transpose(x).reshape(NUM_FIELDS * B)
    tables_t = jnp.transpose(tables, (0, 2, 1))
    h1t = _sc_gather(xt_flat, tables_t)
    return _tc_head(h1t, W.T, b.reshape(1, OUT_DIM))


# speedup vs baseline: 41.1518x; 1.4867x over previous
import functools

import jax
import jax.numpy as jnp
from jax import lax
from jax.experimental import pallas as pl
from jax.experimental.pallas import tpu as pltpu
from jax.experimental.pallas import tpu_sc as plsc

B = 16384
NUM_FIELDS = 26
VOCAB = 100000
PER_FIELD_DIM = 32
HIDDEN = NUM_FIELDS * PER_FIELD_DIM
OUT_DIM = 128

NC = 2
NS = 16
NW = NC * NS
UNITS_PER_TILE = HIDDEN // NW
HALF_B = B // 2


def _sc_gather(xt_flat, tables_t):
    mesh = plsc.VectorSubcoreMesh(core_axis_name="c", subcore_axis_name="s")

    @functools.partial(
        pl.kernel,
        mesh=mesh,
        out_type=jax.ShapeDtypeStruct((HIDDEN, B), jnp.float32),
        scratch_types=[
            pltpu.VMEM((VOCAB,), jnp.float32),
            pltpu.VMEM((HALF_B,), jnp.int32),
            pltpu.VMEM((HALF_B,), jnp.float32),
            pltpu.VMEM((HALF_B,), jnp.float32),
            pltpu.SemaphoreType.DMA,
            pltpu.SemaphoreType.DMA,
            pltpu.SemaphoreType.DMA,
        ],
        compiler_params=pltpu.CompilerParams(needs_layout_passes=False),
    )
    def k(xt_hbm, tab_hbm, out_hbm, slice_v, idx_v, ob0, ob1, ssem, os0, os1):
        wid = lax.axis_index("s") * NC + lax.axis_index("c")
        obufs = (ob0, ob1)
        osems = (os0, os1)
        writes = [None, None]

        def gather_half(ob):
            @plsc.parallel_loop(0, HALF_B, 16, unroll=8)
            def gbody(i):
                ids = idx_v[pl.ds(i, 16)]
                ob[pl.ds(i, 16)] = plsc.load_gather(slice_v, [ids])

        for u in range(UNITS_PER_TILE):
            g = wid * UNITS_PER_TILE + u
            f = g // PER_FIELD_DIM
            d = g % PER_FIELD_DIM
            slice_cp = pltpu.async_copy(tab_hbm.at[f, d, :], slice_v, ssem)
            for h in range(2):
                pltpu.sync_copy(
                    xt_hbm.at[pl.ds(f * B + h * HALF_B, HALF_B)], idx_v)
                if h == 0:
                    slice_cp.wait()
                if writes[h] is not None:
                    writes[h].wait()
                gather_half(obufs[h])
                writes[h] = pltpu.async_copy(
                    obufs[h],
                    out_hbm.at[g, pl.ds(h * HALF_B, HALF_B)],
                    osems[h])
        writes[0].wait()
        writes[1].wait()

    return k(xt_flat, tables_t)


def _tc_head(h1t, wt, b2):
    bm = 1024

    def body(e_ref, w_ref, b_ref, o_ref):
        h = jnp.maximum(e_ref[...], 0.0)
        o_ref[...] = (
            jax.lax.dot_general(
                h, w_ref[...],
                dimension_numbers=(((0,), (0,)), ((), ())),
                preferred_element_type=jnp.float32)
            + b_ref[...]
        )

    return pl.pallas_call(
        body,
        grid=(B // bm,),
        in_specs=[
            pl.BlockSpec((HIDDEN, bm), lambda i: (0, i)),
            pl.BlockSpec((HIDDEN, OUT_DIM), lambda i: (0, 0)),
            pl.BlockSpec((1, OUT_DIM), lambda i: (0, 0)),
        ],
        out_specs=pl.BlockSpec((bm, OUT_DIM), lambda i: (i, 0)),
        out_shape=jax.ShapeDtypeStruct((B, OUT_DIM), jnp.float32),
    )(h1t, wt, b2)


def kernel(x, tables, W, b):
    xt_flat = jnp.transpose(x).reshape(NUM_FIELDS * B)
    tables_t = jnp.transpose(tables, (0, 2, 1))
    h1t = _sc_gather(xt_flat, tables_t)
    return _tc_head(h1t, W.T, b.reshape(1, OUT_DIM))


# speedup vs baseline: 46.9960x; 1.1420x over previous
import functools

import jax
import jax.numpy as jnp
from jax import lax
from jax.experimental import pallas as pl
from jax.experimental.pallas import tpu as pltpu
from jax.experimental.pallas import tpu_sc as plsc

B = 16384
NUM_FIELDS = 26
VOCAB = 100000
PER_FIELD_DIM = 32
HIDDEN = NUM_FIELDS * PER_FIELD_DIM
OUT_DIM = 128

NC = 2
NS = 16
NW = NC * NS
UNITS_PER_TILE = HIDDEN // NW
QTR_B = B // 4


def _sc_gather(xt_flat, tables_t):
    mesh = plsc.VectorSubcoreMesh(core_axis_name="c", subcore_axis_name="s")

    @functools.partial(
        pl.kernel,
        mesh=mesh,
        out_type=jax.ShapeDtypeStruct((HIDDEN, B), jnp.float32),
        scratch_types=[
            pltpu.VMEM((VOCAB,), jnp.float32),
            pltpu.VMEM((B,), jnp.int32),
            pltpu.VMEM((QTR_B,), jnp.float32),
            pltpu.VMEM((QTR_B,), jnp.float32),
            pltpu.SemaphoreType.DMA,
            pltpu.SemaphoreType.DMA,
            pltpu.SemaphoreType.DMA,
        ],
        compiler_params=pltpu.CompilerParams(needs_layout_passes=False),
    )
    def k(xt_hbm, tab_hbm, out_hbm, slice_v, idx_v, ob0, ob1,
          ssem, os0, os1):
        wid = lax.axis_index("s") * NC + lax.axis_index("c")
        obufs = (ob0, ob1)
        osems = (os0, os1)
        writes = [None, None]

        def gather_quarter(ob, q):
            @plsc.parallel_loop(0, QTR_B, 16, unroll=8)
            def gbody(i):
                ids = idx_v[pl.ds(q * QTR_B + i, 16)]
                ob[pl.ds(i, 16)] = plsc.load_gather(slice_v, [ids])

        for u in range(UNITS_PER_TILE):
            g = wid * UNITS_PER_TILE + u
            f = g // PER_FIELD_DIM
            d = g % PER_FIELD_DIM
            slice_cp = pltpu.async_copy(tab_hbm.at[f, d, :], slice_v, ssem)
            if u == 0:
                pltpu.sync_copy(xt_hbm.at[pl.ds(f * B, B)], idx_v)
            else:
                @pl.when(d == 0)
                def _load_idx():
                    pltpu.sync_copy(xt_hbm.at[pl.ds(f * B, B)], idx_v)
            slice_cp.wait()
            for q in range(4):
                if writes[q % 2] is not None:
                    writes[q % 2].wait()
                gather_quarter(obufs[q % 2], q)
                writes[q % 2] = pltpu.async_copy(
                    obufs[q % 2],
                    out_hbm.at[g, pl.ds(q * QTR_B, QTR_B)],
                    osems[q % 2])
        writes[0].wait()
        writes[1].wait()

    return k(xt_flat, tables_t)


def _tc_head(h1t, wt, b2):
    bm = 1024

    def body(e_ref, w_ref, b_ref, o_ref):
        h = jnp.maximum(e_ref[...], 0.0)
        o_ref[...] = (
            jax.lax.dot_general(
                h, w_ref[...],
                dimension_numbers=(((0,), (0,)), ((), ())),
                preferred_element_type=jnp.float32)
            + b_ref[...]
        )

    return pl.pallas_call(
        body,
        grid=(B // bm,),
        in_specs=[
            pl.BlockSpec((HIDDEN, bm), lambda i: (0, i)),
            pl.BlockSpec((HIDDEN, OUT_DIM), lambda i: (0, 0)),
            pl.BlockSpec((1, OUT_DIM), lambda i: (0, 0)),
        ],
        out_specs=pl.BlockSpec((bm, OUT_DIM), lambda i: (i, 0)),
        out_shape=jax.ShapeDtypeStruct((B, OUT_DIM), jnp.float32),
    )(h1t, wt, b2)


def kernel(x, tables, W, b):
    xt_flat = jnp.transpose(x).reshape(NUM_FIELDS * B)
    tables_t = jnp.transpose(tables, (0, 2, 1))
    h1t = _sc_gather(xt_flat, tables_t)
    return _tc_head(h1t, W.T, b.reshape(1, OUT_DIM))


# speedup vs baseline: 47.0608x; 1.0014x over previous
import functools

import jax
import jax.numpy as jnp
from jax import lax
from jax.experimental import pallas as pl
from jax.experimental.pallas import tpu as pltpu
from jax.experimental.pallas import tpu_sc as plsc

B = 16384
NUM_FIELDS = 26
VOCAB = 100000
PER_FIELD_DIM = 32
HIDDEN = NUM_FIELDS * PER_FIELD_DIM
OUT_DIM = 128

NC = 2
NS = 16
NW = NC * NS
UNITS_PER_TILE = HIDDEN // NW
QTR_B = B // 4


def _sc_gather(xt2, tables_t):
    mesh = plsc.VectorSubcoreMesh(core_axis_name="c", subcore_axis_name="s")

    @functools.partial(
        pl.kernel,
        mesh=mesh,
        out_type=jax.ShapeDtypeStruct((HIDDEN, B), jnp.float32),
        scratch_types=[
            pltpu.VMEM((VOCAB,), jnp.float32),
            pltpu.VMEM((B,), jnp.int32),
            pltpu.VMEM((QTR_B,), jnp.float32),
            pltpu.VMEM((QTR_B,), jnp.float32),
            pltpu.SemaphoreType.DMA,
            pltpu.SemaphoreType.DMA,
            pltpu.SemaphoreType.DMA,
        ],
        compiler_params=pltpu.CompilerParams(needs_layout_passes=False),
    )
    def k(xt2_hbm, tab_hbm, out_hbm, slice_v, idx_v, ob0, ob1,
          ssem, os0, os1):
        wid = lax.axis_index("s") * NC + lax.axis_index("c")
        obufs = (ob0, ob1)
        osems = (os0, os1)
        writes = [None, None]

        def gather_quarter(ob, q):
            @plsc.parallel_loop(0, QTR_B, 16, unroll=8)
            def gbody(i):
                ids = idx_v[pl.ds(q * QTR_B + i, 16)]
                ob[pl.ds(i, 16)] = plsc.load_gather(slice_v, [ids])

        for u in range(UNITS_PER_TILE):
            g = wid * UNITS_PER_TILE + u
            f = g // PER_FIELD_DIM
            d = g % PER_FIELD_DIM
            slice_cp = pltpu.async_copy(tab_hbm.at[f, d, :], slice_v, ssem)
            if u == 0:
                pltpu.sync_copy(xt2_hbm.at[f, :], idx_v)
            else:
                @pl.when(d == 0)
                def _load_idx():
                    pltpu.sync_copy(xt2_hbm.at[f, :], idx_v)
            slice_cp.wait()
            for q in range(4):
                if writes[q % 2] is not None:
                    writes[q % 2].wait()
                gather_quarter(obufs[q % 2], q)
                writes[q % 2] = pltpu.async_copy(
                    obufs[q % 2],
                    out_hbm.at[g, pl.ds(q * QTR_B, QTR_B)],
                    osems[q % 2])
        writes[0].wait()
        writes[1].wait()

    return k(xt2, tables_t)


def _tc_head(h1t, wt, b2):
    bm = 1024

    def body(e_ref, w_ref, b_ref, o_ref):
        h = jnp.maximum(e_ref[...], 0.0)
        o_ref[...] = (
            jax.lax.dot_general(
                h, w_ref[...],
                dimension_numbers=(((0,), (0,)), ((), ())),
                preferred_element_type=jnp.float32)
            + b_ref[...]
        )

    return pl.pallas_call(
        body,
        grid=(B // bm,),
        in_specs=[
            pl.BlockSpec((HIDDEN, bm), lambda i: (0, i)),
            pl.BlockSpec((HIDDEN, OUT_DIM), lambda i: (0, 0)),
            pl.BlockSpec((1, OUT_DIM), lambda i: (0, 0)),
        ],
        out_specs=pl.BlockSpec((bm, OUT_DIM), lambda i: (i, 0)),
        out_shape=jax.ShapeDtypeStruct((B, OUT_DIM), jnp.float32),
    )(h1t, wt, b2)


def kernel(x, tables, W, b):
    xt2 = jnp.transpose(x)
    tables_t = jnp.transpose(tables, (0, 2, 1))
    h1t = _sc_gather(xt2, tables_t)
    return _tc_head(h1t, W.T, b.reshape(1, OUT_DIM))
